# Initial kernel scaffold; baseline (speedup 1.0000x reference)
import jax
import jax.numpy as jnp
from jax.experimental import pallas as pl


def kernel(xyz, points, W1, b1, g1, be1, W2, b2, g2, be2, W3, b3, g3, be3):
    raise NotImplementedError("write your pallas kernel here")


# speedup vs baseline: 1.3573x; 1.3573x over previous
import functools

import jax
import jax.numpy as jnp
from jax import lax
from jax.experimental import pallas as pl
from jax.experimental.pallas import tpu as pltpu
from jax.experimental.pallas import tpu_sc as plsc

_B, _N, _C = 8, 2048, 64
_S, _K = 512, 32
_D = 128
_FO = 64
_QW = 16
_ROWS = _B * _S * _K
_EPS = 1e-5
_NW = 32
_CH = 32
_CW = 128


def _fps_body(xs_ref, sel_ref):
    x = xs_ref[0]
    y = xs_ref[1]
    z = xs_ref[2]
    iota_n = lax.broadcasted_iota(jnp.int32, (_B, _N), 1)
    iota_s = lax.broadcasted_iota(jnp.int32, (_B, _S), 1)

    def step(j, carry):
        dist, far, sx, sy, sz = carry
        m = iota_n == far
        cx = jnp.sum(jnp.where(m, x, 0.0), axis=1, keepdims=True)
        cy = jnp.sum(jnp.where(m, y, 0.0), axis=1, keepdims=True)
        cz = jnp.sum(jnp.where(m, z, 0.0), axis=1, keepdims=True)
        rec = iota_s == j
        sx = jnp.where(rec, cx, sx)
        sy = jnp.where(rec, cy, sy)
        sz = jnp.where(rec, cz, sz)
        dx = x - cx
        dy = y - cy
        dz = z - cz
        d = dx * dx + dy * dy + dz * dz
        dist = jnp.minimum(dist, d)
        mx = jnp.max(dist, axis=1, keepdims=True)
        far = jnp.min(jnp.where(dist == mx, iota_n, _N), axis=1, keepdims=True)
        return dist, far, sx, sy, sz

    dist0 = jnp.full((_B, _N), 1e10, jnp.float32)
    far0 = jnp.zeros((_B, 1), jnp.int32)
    zs = jnp.zeros((_B, _S), jnp.float32)
    _, _, sx, sy, sz = lax.fori_loop(0, _S, step, (dist0, far0, zs, zs, zs))
    sel_ref[0] = sx
    sel_ref[1] = sy
    sel_ref[2] = sz


def _fps(xs):
    return pl.pallas_call(
        _fps_body,
        out_shape=jax.ShapeDtypeStruct((3, _B, _S), jnp.float32),
    )(xs)


_QB = 8


def _knn_body(pt_ref, q_ref, idx_ref):
    i = pl.program_id(0)
    b = i // (_S // _QB)
    q = q_ref[...]
    p = pt_ref[0]
    dot = lax.dot_general(q, p, (((1,), (0,)), ((), ())),
                          preferred_element_type=jnp.float32)
    qx = q[:, 0:1]
    qy = q[:, 1:2]
    qz = q[:, 2:3]
    qq = (qx * qx + qy * qy) + qz * qz
    px = p[0:1, :]
    py = p[1:2, :]
    pz = p[2:3, :]
    pp = (px * px + py * py) + pz * pz
    d = (qq + pp) - 2.0 * dot
    iota_n = lax.broadcasted_iota(jnp.int32, (_QB, _N), 1)
    iota_k = lax.broadcasted_iota(jnp.int32, (_QB, _K), 1)

    def step(k, carry):
        d, out = carry
        mn = jnp.min(d, axis=1, keepdims=True)
        am = jnp.min(jnp.where(d == mn, iota_n, _N), axis=1, keepdims=True)
        out = jnp.where(iota_k == k, am, out)
        d = jnp.where(iota_n == am, jnp.inf, d)
        return d, out

    _, out = lax.fori_loop(0, _K, step, (d, jnp.zeros((_QB, _K), jnp.int32)))
    idx_ref[...] = out + b * _N


def _knn(pt, qrows):
    grid = (_B * _S // _QB,)
    return pl.pallas_call(
        _knn_body,
        grid=grid,
        in_specs=[
            pl.BlockSpec((1, 3, _N), lambda i: (i // (_S // _QB), 0, 0)),
            pl.BlockSpec((_QB, 3), lambda i: (i, 0)),
        ],
        out_specs=pl.BlockSpec((_QB, _K), lambda i: (i, 0)),
        out_shape=jax.ShapeDtypeStruct((_B * _S, _K), jnp.int32),
    )(pt, qrows)


def _sc_gather(table, idx3):
    mesh = plsc.VectorSubcoreMesh(core_axis_name="c", subcore_axis_name="s")

    @functools.partial(
        pl.kernel,
        mesh=mesh,
        out_type=jax.ShapeDtypeStruct((_ROWS, _D), jnp.float32),
        scratch_types=[
            pltpu.VMEM((_CH, _CW), jnp.int32),
            pltpu.VMEM((_CW, _D), jnp.float32),
            pltpu.SemaphoreType.DMA,
        ],
    )
    def gk(table_hbm, idx_hbm, out_hbm, idx_v, buf, sem):
        cid = lax.axis_index("c")
        sid = lax.axis_index("s")
        wid = sid * 2 + cid
        pltpu.sync_copy(idx_hbm.at[wid], idx_v)
        base = wid * (_CH * _CW)
        for j in range(_CH):
            pltpu.async_copy(table_hbm.at[idx_v.at[j]], buf, sem).wait()
            pltpu.sync_copy(buf, out_hbm.at[pl.ds(base + j * _CW, _CW)])

    return gk(table, idx3)


_RB = 1024
_INV_ROWS = 1.0 / _ROWS


def _dot(a, b):
    return lax.dot_general(a, b, (((1,), (0,)), ((), ())),
                           preferred_element_type=jnp.float32)


def _accum_stats(i, z, st_ref):
    s = jnp.sum(z, axis=0, keepdims=True)
    s2 = jnp.sum(z * z, axis=0, keepdims=True)

    @pl.when(i == 0)
    def _():
        st_ref[...] = jnp.zeros_like(st_ref)

    st_ref[0:1, :] = st_ref[0:1, :] + s
    st_ref[1:2, :] = st_ref[1:2, :] + s2


def _m1_body(g_ref, qe_ref, w_ref, b_ref, z_ref, st_ref):
    i = pl.program_id(0)
    xb = g_ref[...]
    qfull = jnp.concatenate(
        [qe_ref[...], jnp.zeros((_RB, _D - _QW), jnp.float32)], axis=1)
    z = _dot(xb - qfull, w_ref[...]) + b_ref[...]
    z_ref[...] = z
    _accum_stats(i, z, st_ref)


def _m1(g, qexp, w1p, b1r):
    grid = (_ROWS // _RB,)
    return pl.pallas_call(
        _m1_body,
        grid=grid,
        in_specs=[
            pl.BlockSpec((_RB, _D), lambda i: (i, 0)),
            pl.BlockSpec((_RB, _QW), lambda i: (i, 0)),
            pl.BlockSpec((_D, 64), lambda i: (0, 0)),
            pl.BlockSpec((1, 64), lambda i: (0, 0)),
        ],
        out_specs=[
            pl.BlockSpec((_RB, 64), lambda i: (i, 0)),
            pl.BlockSpec((8, 64), lambda i: (0, 0)),
        ],
        out_shape=[
            jax.ShapeDtypeStruct((_ROWS, 64), jnp.float32),
            jax.ShapeDtypeStruct((8, 64), jnp.float32),
        ],
    )(g, qexp, w1p, b1r)


def _bn_scale_shift(st_ref, gam_ref, bet_ref):
    mean = st_ref[0:1, :] * _INV_ROWS
    ex2 = st_ref[1:2, :] * _INV_ROWS
    var = ex2 - mean * mean
    inv = 1.0 / jnp.sqrt(var + _EPS)
    scale = gam_ref[...] * inv
    shift = bet_ref[...] - mean * scale
    return scale, shift


def _mid_body(z_ref, st_ref, gam_ref, bet_ref, w_ref, b_ref, z2_ref, st2_ref):
    i = pl.program_id(0)
    scale, shift = _bn_scale_shift(st_ref, gam_ref, bet_ref)
    h = jnp.maximum(z_ref[...] * scale + shift, 0.0)
    z2 = _dot(h, w_ref[...]) + b_ref[...]
    z2_ref[...] = z2
    _accum_stats(i, z2, st2_ref)


def _mid(z, st, gam, bet, wt, br, cin, cout):
    grid = (_ROWS // _RB,)
    return pl.pallas_call(
        _mid_body,
        grid=grid,
        in_specs=[
            pl.BlockSpec((_RB, cin), lambda i: (i, 0)),
            pl.BlockSpec((8, cin), lambda i: (0, 0)),
            pl.BlockSpec((1, cin), lambda i: (0, 0)),
            pl.BlockSpec((1, cin), lambda i: (0, 0)),
            pl.BlockSpec((cin, cout), lambda i: (0, 0)),
            pl.BlockSpec((1, cout), lambda i: (0, 0)),
        ],
        out_specs=[
            pl.BlockSpec((_RB, cout), lambda i: (i, 0)),
            pl.BlockSpec((8, cout), lambda i: (0, 0)),
        ],
        out_shape=[
            jax.ShapeDtypeStruct((_ROWS, cout), jnp.float32),
            jax.ShapeDtypeStruct((8, cout), jnp.float32),
        ],
    )(z, st, gam, bet, wt, br)


def _m4_body(z_ref, st_ref, gam_ref, bet_ref, o_ref):
    scale, shift = _bn_scale_shift(st_ref, gam_ref, bet_ref)
    h = jnp.maximum(z_ref[...] * scale + shift, 0.0)
    for s in range(_RB // _K):
        o_ref[s:s + 1, :] = jnp.max(h[s * _K:(s + 1) * _K, :], axis=0,
                                    keepdims=True)


def _m4(z3, st3, gam, bet):
    grid = (_ROWS // _RB,)
    cout = 128
    return pl.pallas_call(
        _m4_body,
        grid=grid,
        in_specs=[
            pl.BlockSpec((_RB, cout), lambda i: (i, 0)),
            pl.BlockSpec((8, cout), lambda i: (0, 0)),
            pl.BlockSpec((1, cout), lambda i: (0, 0)),
            pl.BlockSpec((1, cout), lambda i: (0, 0)),
        ],
        out_specs=pl.BlockSpec((_RB // _K, cout), lambda i: (i, 0)),
        out_shape=jax.ShapeDtypeStruct((_B * _S, cout), jnp.float32),
    )(z3, st3, gam, bet)


def kernel(xyz, points, W1, b1, g1, be1, W2, b2, g2, be2, W3, b3, g3, be3):
    xs = jnp.transpose(xyz, (2, 0, 1))
    sel = _fps(xs)
    qrows = jnp.transpose(sel, (1, 2, 0)).reshape(_B * _S, 3)
    fidx = _knn(jnp.transpose(xyz, (0, 2, 1)), qrows)
    table = jnp.concatenate(
        [xyz, jnp.zeros((_B, _N, _FO - 3), jnp.float32), points],
        axis=2).reshape(_B * _N, _D)
    g = _sc_gather(table, fidx.reshape(_NW, _CH, _CW))
    qexp = jnp.repeat(
        jnp.pad(qrows, ((0, 0), (0, _QW - 3))), _K, axis=0)
    w1p = jnp.zeros((_D, 64), jnp.float32)
    w1p = w1p.at[0:3].set(W1[:, 0:3].T).at[_FO:_FO + _C].set(W1[:, 3:].T)
    z1, st1 = _m1(g, qexp, w1p, b1.reshape(1, -1))
    z2, st2 = _mid(z1, st1, g1.reshape(1, -1), be1.reshape(1, -1),
                   W2.T, b2.reshape(1, -1), 64, 64)
    z3, st3 = _mid(z2, st2, g2.reshape(1, -1), be2.reshape(1, -1),
                   W3.T, b3.reshape(1, -1), 64, 128)
    o = _m4(z3, st3, g3.reshape(1, -1), be3.reshape(1, -1))
    return qrows.reshape(_B, _S, 3), o.reshape(_B, _S, 128)


# speedup vs baseline: 5.6098x; 4.1330x over previous
import functools

import jax
import jax.numpy as jnp
from jax import lax
from jax.experimental import pallas as pl
from jax.experimental.pallas import tpu as pltpu
from jax.experimental.pallas import tpu_sc as plsc

_B, _N, _C = 8, 2048, 64
_S, _K = 512, 32
_D = 128
_FO = 64
_QW = 16
_ROWS = _B * _S * _K
_EPS = 1e-5
_NW = 32
_CH = 32
_CW = 128


def _fps_body(xs_ref, sel_ref):
    x = xs_ref[0]
    y = xs_ref[1]
    z = xs_ref[2]
    iota_n = lax.broadcasted_iota(jnp.int32, (_B, _N), 1)
    iota_s = lax.broadcasted_iota(jnp.int32, (_B, _S), 1)

    def step(j, carry):
        dist, far, sx, sy, sz = carry
        m = iota_n == far
        cx = jnp.sum(jnp.where(m, x, 0.0), axis=1, keepdims=True)
        cy = jnp.sum(jnp.where(m, y, 0.0), axis=1, keepdims=True)
        cz = jnp.sum(jnp.where(m, z, 0.0), axis=1, keepdims=True)
        rec = iota_s == j
        sx = jnp.where(rec, cx, sx)
        sy = jnp.where(rec, cy, sy)
        sz = jnp.where(rec, cz, sz)
        dx = x - cx
        dy = y - cy
        dz = z - cz
        d = dx * dx + dy * dy + dz * dz
        dist = jnp.minimum(dist, d)
        mx = jnp.max(dist, axis=1, keepdims=True)
        far = jnp.min(jnp.where(dist == mx, iota_n, _N), axis=1, keepdims=True)
        return dist, far, sx, sy, sz

    dist0 = jnp.full((_B, _N), 1e10, jnp.float32)
    far0 = jnp.zeros((_B, 1), jnp.int32)
    zs = jnp.zeros((_B, _S), jnp.float32)
    _, _, sx, sy, sz = lax.fori_loop(0, _S, step, (dist0, far0, zs, zs, zs))
    sel_ref[0] = sx
    sel_ref[1] = sy
    sel_ref[2] = sz


def _fps(xs):
    return pl.pallas_call(
        _fps_body,
        out_shape=jax.ShapeDtypeStruct((3, _B, _S), jnp.float32),
    )(xs)


_QB = 64


def _knn_body(pt_ref, q_ref, idx_ref):
    i = pl.program_id(0)
    b = i // (_S // _QB)
    q = q_ref[...]
    p = pt_ref[0]
    dot = lax.dot_general(q, p, (((1,), (0,)), ((), ())),
                          preferred_element_type=jnp.float32)
    qx = q[:, 0:1]
    qy = q[:, 1:2]
    qz = q[:, 2:3]
    qq = (qx * qx + qy * qy) + qz * qz
    px = p[0:1, :]
    py = p[1:2, :]
    pz = p[2:3, :]
    pp = (px * px + py * py) + pz * pz
    d = (qq + pp) - 2.0 * dot
    iota_n = lax.broadcasted_iota(jnp.int32, (_QB, _N), 1)
    iota_k = lax.broadcasted_iota(jnp.int32, (_QB, _K), 1)

    s = lax.bitcast_convert_type(d, jnp.int32)
    fsec = lax.bitcast_convert_type(
        ((s & jnp.int32(~2047)) | iota_n) + jnp.int32(0x10000000), jnp.float32)
    inf = jnp.float32(jnp.inf)

    def step(k, carry):
        d, out = carry
        mn = jnp.min(d, axis=1, keepdims=True)
        sel = jnp.min(jnp.where(d == mn, fsec, inf), axis=1, keepdims=True)
        out = jnp.where(iota_k == k, lax.bitcast_convert_type(sel, jnp.int32),
                        out)
        d = jnp.where(fsec == sel, inf, d)
        return d, out

    _, out = lax.fori_loop(0, _K, step, (d, jnp.zeros((_QB, _K), jnp.int32)))
    idx_ref[...] = (out & jnp.int32(2047)) + b * _N


def _knn(pt, qrows):
    grid = (_B * _S // _QB,)
    return pl.pallas_call(
        _knn_body,
        grid=grid,
        in_specs=[
            pl.BlockSpec((1, 3, _N), lambda i, qb=_QB: (i // (_S // qb), 0, 0)),
            pl.BlockSpec((_QB, 3), lambda i: (i, 0)),
        ],
        out_specs=pl.BlockSpec((_QB, _K), lambda i: (i, 0)),
        out_shape=jax.ShapeDtypeStruct((_B * _S, _K), jnp.int32),
    )(pt, qrows)


def _sc_gather(table, idx3):
    mesh = plsc.VectorSubcoreMesh(core_axis_name="c", subcore_axis_name="s")

    @functools.partial(
        pl.kernel,
        mesh=mesh,
        out_type=jax.ShapeDtypeStruct((_ROWS, _D), jnp.float32),
        scratch_types=[
            pltpu.VMEM((_CH, _CW), jnp.int32),
            pltpu.VMEM((_CW, _D), jnp.float32),
            pltpu.SemaphoreType.DMA,
        ],
    )
    def gk(table_hbm, idx_hbm, out_hbm, idx_v, buf, sem):
        cid = lax.axis_index("c")
        sid = lax.axis_index("s")
        wid = sid * 2 + cid
        pltpu.sync_copy(idx_hbm.at[wid], idx_v)
        base = wid * (_CH * _CW)
        for j in range(_CH):
            pltpu.async_copy(table_hbm.at[idx_v.at[j]], buf, sem).wait()
            pltpu.sync_copy(buf, out_hbm.at[pl.ds(base + j * _CW, _CW)])

    return gk(table, idx3)


_RB = 1024
_INV_ROWS = 1.0 / _ROWS


def _dot(a, b):
    return lax.dot_general(a, b, (((1,), (0,)), ((), ())),
                           preferred_element_type=jnp.float32)


def _accum_stats(i, z, st_ref):
    s = jnp.sum(z, axis=0, keepdims=True)
    s2 = jnp.sum(z * z, axis=0, keepdims=True)

    @pl.when(i == 0)
    def _():
        st_ref[...] = jnp.zeros_like(st_ref)

    st_ref[0:1, :] = st_ref[0:1, :] + s
    st_ref[1:2, :] = st_ref[1:2, :] + s2


def _m1_body(g_ref, qe_ref, w_ref, b_ref, z_ref, st_ref):
    i = pl.program_id(0)
    xb = g_ref[...]
    qfull = jnp.concatenate(
        [qe_ref[...], jnp.zeros((_RB, _D - _QW), jnp.float32)], axis=1)
    z = _dot(xb - qfull, w_ref[...]) + b_ref[...]
    z_ref[...] = z
    _accum_stats(i, z, st_ref)


def _m1(g, qexp, w1p, b1r):
    grid = (_ROWS // _RB,)
    return pl.pallas_call(
        _m1_body,
        grid=grid,
        in_specs=[
            pl.BlockSpec((_RB, _D), lambda i: (i, 0)),
            pl.BlockSpec((_RB, _QW), lambda i: (i, 0)),
            pl.BlockSpec((_D, 64), lambda i: (0, 0)),
            pl.BlockSpec((1, 64), lambda i: (0, 0)),
        ],
        out_specs=[
            pl.BlockSpec((_RB, 64), lambda i: (i, 0)),
            pl.BlockSpec((8, 64), lambda i: (0, 0)),
        ],
        out_shape=[
            jax.ShapeDtypeStruct((_ROWS, 64), jnp.float32),
            jax.ShapeDtypeStruct((8, 64), jnp.float32),
        ],
    )(g, qexp, w1p, b1r)


def _bn_scale_shift(st_ref, gam_ref, bet_ref):
    mean = st_ref[0:1, :] * _INV_ROWS
    ex2 = st_ref[1:2, :] * _INV_ROWS
    var = ex2 - mean * mean
    inv = 1.0 / jnp.sqrt(var + _EPS)
    scale = gam_ref[...] * inv
    shift = bet_ref[...] - mean * scale
    return scale, shift


def _mid_body(z_ref, st_ref, gam_ref, bet_ref, w_ref, b_ref, z2_ref, st2_ref):
    i = pl.program_id(0)
    scale, shift = _bn_scale_shift(st_ref, gam_ref, bet_ref)
    h = jnp.maximum(z_ref[...] * scale + shift, 0.0)
    z2 = _dot(h, w_ref[...]) + b_ref[...]
    z2_ref[...] = z2
    _accum_stats(i, z2, st2_ref)


def _mid(z, st, gam, bet, wt, br, cin, cout):
    grid = (_ROWS // _RB,)
    return pl.pallas_call(
        _mid_body,
        grid=grid,
        in_specs=[
            pl.BlockSpec((_RB, cin), lambda i: (i, 0)),
            pl.BlockSpec((8, cin), lambda i: (0, 0)),
            pl.BlockSpec((1, cin), lambda i: (0, 0)),
            pl.BlockSpec((1, cin), lambda i: (0, 0)),
            pl.BlockSpec((cin, cout), lambda i: (0, 0)),
            pl.BlockSpec((1, cout), lambda i: (0, 0)),
        ],
        out_specs=[
            pl.BlockSpec((_RB, cout), lambda i: (i, 0)),
            pl.BlockSpec((8, cout), lambda i: (0, 0)),
        ],
        out_shape=[
            jax.ShapeDtypeStruct((_ROWS, cout), jnp.float32),
            jax.ShapeDtypeStruct((8, cout), jnp.float32),
        ],
    )(z, st, gam, bet, wt, br)


def _m4_body(z_ref, st_ref, gam_ref, bet_ref, o_ref):
    scale, shift = _bn_scale_shift(st_ref, gam_ref, bet_ref)
    h = jnp.maximum(z_ref[...] * scale + shift, 0.0)
    for s in range(_RB // _K):
        o_ref[s:s + 1, :] = jnp.max(h[s * _K:(s + 1) * _K, :], axis=0,
                                    keepdims=True)


def _m4(z3, st3, gam, bet):
    grid = (_ROWS // _RB,)
    cout = 128
    return pl.pallas_call(
        _m4_body,
        grid=grid,
        in_specs=[
            pl.BlockSpec((_RB, cout), lambda i: (i, 0)),
            pl.BlockSpec((8, cout), lambda i: (0, 0)),
            pl.BlockSpec((1, cout), lambda i: (0, 0)),
            pl.BlockSpec((1, cout), lambda i: (0, 0)),
        ],
        out_specs=pl.BlockSpec((_RB // _K, cout), lambda i: (i, 0)),
        out_shape=jax.ShapeDtypeStruct((_B * _S, cout), jnp.float32),
    )(z3, st3, gam, bet)


def kernel(xyz, points, W1, b1, g1, be1, W2, b2, g2, be2, W3, b3, g3, be3):
    xs = jnp.transpose(xyz, (2, 0, 1))
    sel = _fps(xs)
    qrows = jnp.transpose(sel, (1, 2, 0)).reshape(_B * _S, 3)
    fidx = _knn(jnp.transpose(xyz, (0, 2, 1)), qrows)
    table = jnp.concatenate(
        [xyz, jnp.zeros((_B, _N, _FO - 3), jnp.float32), points],
        axis=2).reshape(_B * _N, _D)
    g = _sc_gather(table, fidx.reshape(_NW, _CH, _CW))
    qexp = jnp.repeat(
        jnp.pad(qrows, ((0, 0), (0, _QW - 3))), _K, axis=0)
    w1p = jnp.zeros((_D, 64), jnp.float32)
    w1p = w1p.at[0:3].set(W1[:, 0:3].T).at[_FO:_FO + _C].set(W1[:, 3:].T)
    z1, st1 = _m1(g, qexp, w1p, b1.reshape(1, -1))
    z2, st2 = _mid(z1, st1, g1.reshape(1, -1), be1.reshape(1, -1),
                   W2.T, b2.reshape(1, -1), 64, 64)
    z3, st3 = _mid(z2, st2, g2.reshape(1, -1), be2.reshape(1, -1),
                   W3.T, b3.reshape(1, -1), 64, 128)
    o = _m4(z3, st3, g3.reshape(1, -1), be3.reshape(1, -1))
    return qrows.reshape(_B, _S, 3), o.reshape(_B, _S, 128)


# speedup vs baseline: 6.0874x; 1.0851x over previous
import functools

import jax
import jax.numpy as jnp
from jax import lax
from jax.experimental import pallas as pl
from jax.experimental.pallas import tpu as pltpu
from jax.experimental.pallas import tpu_sc as plsc

_B, _N, _C = 8, 2048, 64
_S, _K = 512, 32
_D = 128
_FO = 64
_QW = 16
_ROWS = _B * _S * _K
_EPS = 1e-5
_NW = 32
_CH = 32
_CW = 128


def _fps_body(xs_ref, sel_ref):
    x = xs_ref[0]
    y = xs_ref[1]
    z = xs_ref[2]
    iota_f = lax.broadcasted_iota(jnp.int32, (_B, _N), 1).astype(jnp.float32)
    iota_s = lax.broadcasted_iota(jnp.int32, (_B, _S), 1)
    inf = jnp.float32(jnp.inf)

    def step(j, carry):
        dist, far, sx, sy, sz = carry
        m = iota_f == far
        cx = jnp.sum(jnp.where(m, x, 0.0), axis=1, keepdims=True)
        cy = jnp.sum(jnp.where(m, y, 0.0), axis=1, keepdims=True)
        cz = jnp.sum(jnp.where(m, z, 0.0), axis=1, keepdims=True)
        rec = iota_s == j
        sx = jnp.where(rec, cx, sx)
        sy = jnp.where(rec, cy, sy)
        sz = jnp.where(rec, cz, sz)
        dx = x - cx
        dy = y - cy
        dz = z - cz
        d = dx * dx + dy * dy + dz * dz
        dist = jnp.minimum(dist, d)
        mx = jnp.max(dist, axis=1, keepdims=True)
        far = jnp.min(jnp.where(dist == mx, iota_f, inf), axis=1, keepdims=True)
        return dist, far, sx, sy, sz

    dist0 = jnp.full((_B, _N), 1e10, jnp.float32)
    far0 = jnp.zeros((_B, 1), jnp.float32)
    zs = jnp.zeros((_B, _S), jnp.float32)
    _, _, sx, sy, sz = lax.fori_loop(0, _S, step, (dist0, far0, zs, zs, zs))
    sel_ref[0] = sx
    sel_ref[1] = sy
    sel_ref[2] = sz


def _fps(xs):
    return pl.pallas_call(
        _fps_body,
        out_shape=jax.ShapeDtypeStruct((3, _B, _S), jnp.float32),
    )(xs)


_QB = 64


def _knn_body(pt_ref, q_ref, idx_ref):
    i = pl.program_id(0)
    b = i // (_S // _QB)
    q = q_ref[...]
    p = pt_ref[0]
    dot = lax.dot_general(q, p, (((1,), (0,)), ((), ())),
                          preferred_element_type=jnp.float32)
    qx = q[:, 0:1]
    qy = q[:, 1:2]
    qz = q[:, 2:3]
    qq = (qx * qx + qy * qy) + qz * qz
    px = p[0:1, :]
    py = p[1:2, :]
    pz = p[2:3, :]
    pp = (px * px + py * py) + pz * pz
    d = (qq + pp) - 2.0 * dot
    iota_n = lax.broadcasted_iota(jnp.int32, (_QB, _N), 1)
    iota_k = lax.broadcasted_iota(jnp.int32, (_QB, _K), 1)

    iota_f = iota_n.astype(jnp.float32)
    inf = jnp.float32(jnp.inf)

    def step(k, carry):
        d, out = carry
        mn = jnp.min(d, axis=1, keepdims=True)
        sel = jnp.min(jnp.where(d == mn, iota_f, inf), axis=1, keepdims=True)
        out = jnp.where(iota_k == k, sel.astype(jnp.int32), out)
        d = jnp.where(iota_f == sel, inf, d)
        return d, out

    _, out = lax.fori_loop(0, _K, step, (d, jnp.zeros((_QB, _K), jnp.int32)))
    idx_ref[...] = out + b * _N


def _knn(pt, qrows):
    grid = (_B * _S // _QB,)
    return pl.pallas_call(
        _knn_body,
        grid=grid,
        in_specs=[
            pl.BlockSpec((1, 3, _N), lambda i, qb=_QB: (i // (_S // qb), 0, 0)),
            pl.BlockSpec((_QB, 3), lambda i: (i, 0)),
        ],
        out_specs=pl.BlockSpec((_QB, _K), lambda i: (i, 0)),
        out_shape=jax.ShapeDtypeStruct((_B * _S, _K), jnp.int32),
    )(pt, qrows)


def _sc_gather(table, idx3):
    mesh = plsc.VectorSubcoreMesh(core_axis_name="c", subcore_axis_name="s")

    @functools.partial(
        pl.kernel,
        mesh=mesh,
        out_type=jax.ShapeDtypeStruct((_ROWS, _D), jnp.float32),
        scratch_types=[
            pltpu.VMEM((_CH, _CW), jnp.int32),
            pltpu.VMEM((_CW, _D), jnp.float32),
            pltpu.VMEM((_CW, _D), jnp.float32),
            pltpu.SemaphoreType.DMA,
            pltpu.SemaphoreType.DMA,
            pltpu.SemaphoreType.DMA,
            pltpu.SemaphoreType.DMA,
        ],
    )
    def gk(table_hbm, idx_hbm, out_hbm, idx_v, buf0, buf1, g0, g1, c0, c1):
        cid = lax.axis_index("c")
        sid = lax.axis_index("s")
        wid = sid * 2 + cid
        pltpu.sync_copy(idx_hbm.at[wid], idx_v)
        base = wid * (_CH * _CW)
        bufs = (buf0, buf1)
        gsems = (g0, g1)
        csems = (c0, c1)
        gather = [None, None]
        copy = [None, None]
        gather[0] = pltpu.async_copy(table_hbm.at[idx_v.at[0]], buf0, g0)
        for j in range(_CH):
            cur = j % 2
            nxt = (j + 1) % 2
            if j + 1 < _CH:
                if copy[nxt] is not None:
                    copy[nxt].wait()
                gather[nxt] = pltpu.async_copy(
                    table_hbm.at[idx_v.at[j + 1]], bufs[nxt], gsems[nxt])
            gather[cur].wait()
            copy[cur] = pltpu.async_copy(
                bufs[cur], out_hbm.at[pl.ds(base + j * _CW, _CW)], csems[cur])
        copy[0].wait()
        copy[1].wait()

    return gk(table, idx3)


_RB = 1024
_INV_ROWS = 1.0 / _ROWS


def _dot(a, b):
    return lax.dot_general(a, b, (((1,), (0,)), ((), ())),
                           preferred_element_type=jnp.float32)


def _accum_stats(i, z, st_ref):
    s = jnp.sum(z, axis=0, keepdims=True)
    s2 = jnp.sum(z * z, axis=0, keepdims=True)

    @pl.when(i == 0)
    def _():
        st_ref[...] = jnp.zeros_like(st_ref)

    st_ref[0:1, :] = st_ref[0:1, :] + s
    st_ref[1:2, :] = st_ref[1:2, :] + s2


def _m1_body(g_ref, qe_ref, w_ref, b_ref, z_ref, st_ref):
    i = pl.program_id(0)
    xb = g_ref[...]
    qfull = jnp.concatenate(
        [qe_ref[...], jnp.zeros((_RB, _D - _QW), jnp.float32)], axis=1)
    z = _dot(xb - qfull, w_ref[...]) + b_ref[...]
    z_ref[...] = z
    _accum_stats(i, z, st_ref)


def _m1(g, qexp, w1p, b1r):
    grid = (_ROWS // _RB,)
    return pl.pallas_call(
        _m1_body,
        grid=grid,
        in_specs=[
            pl.BlockSpec((_RB, _D), lambda i: (i, 0)),
            pl.BlockSpec((_RB, _QW), lambda i: (i, 0)),
            pl.BlockSpec((_D, 64), lambda i: (0, 0)),
            pl.BlockSpec((1, 64), lambda i: (0, 0)),
        ],
        out_specs=[
            pl.BlockSpec((_RB, 64), lambda i: (i, 0)),
            pl.BlockSpec((8, 64), lambda i: (0, 0)),
        ],
        out_shape=[
            jax.ShapeDtypeStruct((_ROWS, 64), jnp.float32),
            jax.ShapeDtypeStruct((8, 64), jnp.float32),
        ],
    )(g, qexp, w1p, b1r)


def _bn_scale_shift(st_ref, gam_ref, bet_ref):
    mean = st_ref[0:1, :] * _INV_ROWS
    ex2 = st_ref[1:2, :] * _INV_ROWS
    var = ex2 - mean * mean
    inv = 1.0 / jnp.sqrt(var + _EPS)
    scale = gam_ref[...] * inv
    shift = bet_ref[...] - mean * scale
    return scale, shift


def _mid_body(z_ref, st_ref, gam_ref, bet_ref, w_ref, b_ref, z2_ref, st2_ref):
    i = pl.program_id(0)
    scale, shift = _bn_scale_shift(st_ref, gam_ref, bet_ref)
    h = jnp.maximum(z_ref[...] * scale + shift, 0.0)
    z2 = _dot(h, w_ref[...]) + b_ref[...]
    z2_ref[...] = z2
    _accum_stats(i, z2, st2_ref)


def _m3_body(z_ref, st_ref, gam_ref, bet_ref, w_ref, b_ref, zp_ref, st2_ref):
    i = pl.program_id(0)
    scale, shift = _bn_scale_shift(st_ref, gam_ref, bet_ref)
    h = jnp.maximum(z_ref[...] * scale + shift, 0.0)
    z2 = _dot(h, w_ref[...]) + b_ref[...]
    _accum_stats(i, z2, st2_ref)
    for s in range(_RB // _K):
        zp_ref[s:s + 1, :] = jnp.max(z2[s * _K:(s + 1) * _K, :], axis=0,
                                     keepdims=True)


def _m3(z, st, gam, bet, wt, br, cin, cout):
    grid = (_ROWS // _RB,)
    return pl.pallas_call(
        _m3_body,
        grid=grid,
        in_specs=[
            pl.BlockSpec((_RB, cin), lambda i: (i, 0)),
            pl.BlockSpec((8, cin), lambda i: (0, 0)),
            pl.BlockSpec((1, cin), lambda i: (0, 0)),
            pl.BlockSpec((1, cin), lambda i: (0, 0)),
            pl.BlockSpec((cin, cout), lambda i: (0, 0)),
            pl.BlockSpec((1, cout), lambda i: (0, 0)),
        ],
        out_specs=[
            pl.BlockSpec((_RB // _K, cout), lambda i: (i, 0)),
            pl.BlockSpec((8, cout), lambda i: (0, 0)),
        ],
        out_shape=[
            jax.ShapeDtypeStruct((_ROWS // _K, cout), jnp.float32),
            jax.ShapeDtypeStruct((8, cout), jnp.float32),
        ],
    )(z, st, gam, bet, wt, br)


def _mid(z, st, gam, bet, wt, br, cin, cout):
    grid = (_ROWS // _RB,)
    return pl.pallas_call(
        _mid_body,
        grid=grid,
        in_specs=[
            pl.BlockSpec((_RB, cin), lambda i: (i, 0)),
            pl.BlockSpec((8, cin), lambda i: (0, 0)),
            pl.BlockSpec((1, cin), lambda i: (0, 0)),
            pl.BlockSpec((1, cin), lambda i: (0, 0)),
            pl.BlockSpec((cin, cout), lambda i: (0, 0)),
            pl.BlockSpec((1, cout), lambda i: (0, 0)),
        ],
        out_specs=[
            pl.BlockSpec((_RB, cout), lambda i: (i, 0)),
            pl.BlockSpec((8, cout), lambda i: (0, 0)),
        ],
        out_shape=[
            jax.ShapeDtypeStruct((_ROWS, cout), jnp.float32),
            jax.ShapeDtypeStruct((8, cout), jnp.float32),
        ],
    )(z, st, gam, bet, wt, br)


def _m4_body(z_ref, st_ref, gam_ref, bet_ref, o_ref):
    scale, shift = _bn_scale_shift(st_ref, gam_ref, bet_ref)
    o_ref[...] = jnp.maximum(z_ref[...] * scale + shift, 0.0)


def _m4(zp, st3, gam, bet):
    cout = 128
    rb = 512
    grid = (_B * _S // rb,)
    return pl.pallas_call(
        _m4_body,
        grid=grid,
        in_specs=[
            pl.BlockSpec((rb, cout), lambda i: (i, 0)),
            pl.BlockSpec((8, cout), lambda i: (0, 0)),
            pl.BlockSpec((1, cout), lambda i: (0, 0)),
            pl.BlockSpec((1, cout), lambda i: (0, 0)),
        ],
        out_specs=pl.BlockSpec((rb, cout), lambda i: (i, 0)),
        out_shape=jax.ShapeDtypeStruct((_B * _S, cout), jnp.float32),
    )(zp, st3, gam, bet)


def kernel(xyz, points, W1, b1, g1, be1, W2, b2, g2, be2, W3, b3, g3, be3):
    xs = jnp.transpose(xyz, (2, 0, 1))
    sel = _fps(xs)
    qrows = jnp.transpose(sel, (1, 2, 0)).reshape(_B * _S, 3)
    fidx = _knn(jnp.transpose(xyz, (0, 2, 1)), qrows)
    table = jnp.concatenate(
        [xyz, jnp.zeros((_B, _N, _FO - 3), jnp.float32), points],
        axis=2).reshape(_B * _N, _D)
    g = _sc_gather(table, fidx.reshape(_NW, _CH, _CW))
    qexp = jnp.repeat(
        jnp.pad(qrows, ((0, 0), (0, _QW - 3))), _K, axis=0)
    w1p = jnp.zeros((_D, 64), jnp.float32)
    w1p = w1p.at[0:3].set(W1[:, 0:3].T).at[_FO:_FO + _C].set(W1[:, 3:].T)
    z1, st1 = _m1(g, qexp, w1p, b1.reshape(1, -1))
    z2, st2 = _mid(z1, st1, g1.reshape(1, -1), be1.reshape(1, -1),
                   W2.T, b2.reshape(1, -1), 64, 64)
    zp, st3 = _m3(z2, st2, g2.reshape(1, -1), be2.reshape(1, -1),
                  W3.T, b3.reshape(1, -1), 64, 128)
    o = _m4(zp, st3, g3.reshape(1, -1), be3.reshape(1, -1))
    return qrows.reshape(_B, _S, 3), o.reshape(_B, _S, 128)
